# baseline (speedup 1.0000x reference)
import functools

import jax
import jax.numpy as jnp
from jax import lax
from jax.experimental import pallas as pl
from jax.experimental.pallas import tpu as pltpu
from jax.experimental.pallas import tpu_sc as plsc

N = 10000
E = 320000
D = 128

NC = 2
NS = 16
NW = NC * NS
EPW = E // NW
CH = 128
NCHUNK = E // CH
JFULL = NCHUNK // NW
JREM = NCHUNK - JFULL * NW
NP = 10240
RPT = NP // NS


def _sc_partial_segment_sum(x, ei5, zeros_slab):
    mesh = plsc.VectorSubcoreMesh(
        core_axis_name="c", subcore_axis_name="s", num_cores=NC,
        num_subcores=NS)

    @functools.partial(
        pl.kernel,
        out_type=jax.ShapeDtypeStruct((NC, NP, D), jnp.float32),
        mesh=mesh,
        scratch_types=dict(
            sidx=[pltpu.VMEM((1, CH), jnp.int32) for _ in range(4)],
            didx=[pltpu.VMEM((1, CH), jnp.int32) for _ in range(4)],
            rows=[pltpu.VMEM((CH, D), jnp.float32) for _ in range(2)],
            acc=pltpu.VMEM_SHARED((NP, D), jnp.float32),
            gsem=[pltpu.SemaphoreType.DMA for _ in range(2)],
            ssem=[pltpu.SemaphoreType.DMA for _ in range(2)],
            isem=[pltpu.SemaphoreType.DMA for _ in range(4)],
        ),
    )
    def sc_kernel(x_hbm, ei_hbm, zero_hbm, out_hbm,
                  sidx, didx, rows, acc, gsem, ssem, isem):
        src_flat = ei_hbm.at[0]
        dst_flat = ei_hbm.at[1]
        c = lax.axis_index("c")
        s = lax.axis_index("s")
        wid = c * NS + s
        jtot = jnp.where(wid < JREM, JFULL + 1, JFULL)

        def fetch_idx(j, k):
            off = (wid + j * NW) * CH
            pltpu.async_copy(src_flat.at[pl.ds(off, CH)],
                             sidx[k].at[0], isem[k])
            pltpu.async_copy(dst_flat.at[pl.ds(off, CH)],
                             didx[k].at[0], isem[k])

        def wait_idx(k):
            pltpu.make_async_copy(src_flat.at[pl.ds(wid * CH, CH)],
                                  sidx[k].at[0], isem[k]).wait()
            pltpu.make_async_copy(dst_flat.at[pl.ds(wid * CH, CH)],
                                  didx[k].at[0], isem[k]).wait()

        def gather(p, k):
            pltpu.async_copy(x_hbm.at[sidx[k].at[0]], rows[p], gsem[p])

        def wait_gather(p):
            pltpu.make_async_copy(x_hbm.at[sidx[0].at[0]], rows[p],
                                  gsem[p]).wait()

        def scatter(p, k):
            pltpu.async_copy(rows[p], acc.at[didx[k].at[0]], ssem[p],
                             add=True)

        def wait_scatter(p):
            pltpu.make_async_copy(zero_hbm.at[pl.ds(0, CH)], rows[p],
                                  ssem[p]).wait()

        pltpu.sync_copy(zero_hbm, acc.at[pl.ds(s * RPT, RPT)])
        plsc.subcore_barrier()

        fetch_idx(0, 0)
        fetch_idx(1, 1)
        wait_idx(0)
        gather(0, 0)
        wait_idx(1)
        gather(1, 1)
        wait_gather(0)
        scatter(0, 0)
        fetch_idx(2, 2)
        wait_gather(1)
        scatter(1, 1)
        wait_scatter(0)
        fetch_idx(3, 3)
        wait_idx(2)
        gather(0, 2)

        def body(g, carry):
            for i in range(4):
                j = 2 + 4 * g + i
                p = i % 2
                k = (2 + i) % 4
                wait_gather(p)
                scatter(p, k)
                wait_scatter(1 - p)
                jn = j + 1

                @pl.when(j + 2 < jtot)
                def _():
                    fetch_idx(j + 2, (k + 2) % 4)

                @pl.when(jn < jtot)
                def _():
                    wait_idx((k + 1) % 4)
                    gather(1 - p, (k + 1) % 4)
            return carry

        lax.fori_loop(0, (JFULL - 2) // 4, body, 0)
        @pl.when(wid < JREM)
        def _():
            wait_gather(JFULL % 2)
            scatter(JFULL % 2, JFULL % 4)
        wait_scatter((JFULL - 1) % 2)

        @pl.when(wid < JREM)
        def _():
            wait_scatter(JFULL % 2)
        plsc.subcore_barrier()
        pltpu.sync_copy(acc.at[pl.ds(s * RPT, RPT)],
                        out_hbm.at[c].at[pl.ds(s * RPT, RPT)])

    return sc_kernel(x, ei5, zeros_slab)


BN = 2000


def _tc_mlp_body(p_ref, x_ref, w1_ref, b1_ref, w2_ref, b2_ref, e_ref,
                 o_ref):
    scale = 1.0 + e_ref[0, 0]
    y = p_ref[0] + p_ref[1] + scale * x_ref[...]
    h = jnp.maximum(
        jnp.dot(y, w1_ref[...], preferred_element_type=jnp.float32)
        + b1_ref[...], 0.0)
    o_ref[...] = (
        jnp.dot(h, w2_ref[...], preferred_element_type=jnp.float32)
        + b2_ref[...])


def _tc_mlp(partial, x, W1, b1, W2, b2, eps):
    grid = (N // BN,)
    return pl.pallas_call(
        _tc_mlp_body,
        grid=grid,
        in_specs=[
            pl.BlockSpec((NC, BN, D), lambda i: (0, i, 0)),
            pl.BlockSpec((BN, D), lambda i: (i, 0)),
            pl.BlockSpec((D, D), lambda i: (0, 0)),
            pl.BlockSpec((1, D), lambda i: (0, 0)),
            pl.BlockSpec((D, D), lambda i: (0, 0)),
            pl.BlockSpec((1, D), lambda i: (0, 0)),
            pl.BlockSpec((1, 1), lambda i: (0, 0)),
        ],
        out_specs=pl.BlockSpec((BN, D), lambda i: (i, 0)),
        out_shape=jax.ShapeDtypeStruct((N, D), jnp.float32),
    )(partial, x, W1, b1.reshape(1, D), W2, b2.reshape(1, D),
      eps.reshape(1, 1))


def kernel(x, edge_index, W1, b1, W2, b2, eps):
    zeros_slab = jnp.zeros((RPT, D), jnp.float32)
    partial = _sc_partial_segment_sum(x, edge_index, zeros_slab)
    return _tc_mlp(partial, x, W1, b1, W2, b2, eps)


# speedup vs baseline: 1.0956x; 1.0956x over previous
import functools

import jax
import jax.numpy as jnp
from jax import lax
from jax.experimental import pallas as pl
from jax.experimental.pallas import tpu as pltpu
from jax.experimental.pallas import tpu_sc as plsc

N = 10000
E = 320000
D = 128

NC = 2
NS = 16
NW = NC * NS
EPW = E // NW
CH = 40
NCHUNK = EPW // CH
NBUF = 5
NROUND = NCHUNK // NBUF
NP = 10240
RPT = NP // NS


def _sc_partial_segment_sum(x, ei5, zeros_slab):
    mesh = plsc.VectorSubcoreMesh(
        core_axis_name="c", subcore_axis_name="s", num_cores=NC,
        num_subcores=NS)

    @functools.partial(
        pl.kernel,
        out_type=jax.ShapeDtypeStruct((NC, NP, D), jnp.float32),
        mesh=mesh,
        scratch_types=dict(
            sbuf=[pltpu.VMEM((NBUF, CH), jnp.int32) for _ in range(2)],
            dbuf=[pltpu.VMEM((NBUF, CH), jnp.int32) for _ in range(2)],
            rows=[pltpu.VMEM((CH, D), jnp.float32) for _ in range(NBUF)],
            acc=pltpu.VMEM_SHARED((NP, D), jnp.float32),
            gsem=[pltpu.SemaphoreType.DMA for _ in range(NBUF)],
            ssem=[pltpu.SemaphoreType.DMA for _ in range(NBUF)],
            isem=[pltpu.SemaphoreType.DMA for _ in range(2)],
        ),
    )
    def sc_kernel(x_hbm, ei_hbm, zero_hbm, out_hbm,
                  sbuf, dbuf, rows, acc, gsem, ssem, isem):
        c = lax.axis_index("c")
        s = lax.axis_index("s")
        wid = c * NS + s
        ebase = wid * EPW

        def fetch_round(rr, sb, db, sem):
            for b in range(NBUF):
                off = ebase + rr * (NBUF * CH) + b * CH
                pltpu.async_copy(ei_hbm.at[pl.ds(off, CH)],
                                 sb.at[b], sem)
                pltpu.async_copy(ei_hbm.at[pl.ds(E + off, CH)],
                                 db.at[b], sem)

        def drain_round(sb, db, sem):
            for b in range(NBUF):
                pltpu.make_async_copy(ei_hbm.at[pl.ds(ebase, CH)],
                                      sb.at[b], sem).wait()
                pltpu.make_async_copy(ei_hbm.at[pl.ds(ebase, CH)],
                                      db.at[b], sem).wait()

        pltpu.sync_copy(zero_hbm, acc.at[pl.ds(s * RPT, RPT)])
        plsc.subcore_barrier()

        fetch_round(0, sbuf[0], dbuf[0], isem[0])
        fetch_round(1, sbuf[1], dbuf[1], isem[1])
        drain_round(sbuf[0], dbuf[0], isem[0])
        for b in range(NBUF):
            pltpu.async_copy(x_hbm.at[sbuf[0].at[b]], rows[b], gsem[b])

        def body(g, carry):
            for par in range(2):
                r = 2 * g + par
                sb, db = sbuf[par], dbuf[par]
                so, do = sbuf[1 - par], dbuf[1 - par]
                scat = []
                for b in range(NBUF):
                    pltpu.make_async_copy(
                        x_hbm.at[sb.at[b]], rows[b], gsem[b]).wait()
                    scat.append(pltpu.async_copy(
                        rows[b], acc.at[db.at[b]], ssem[b], add=True))
                @pl.when(r < NROUND - 1)
                def _():
                    drain_round(so, do, isem[1 - par])
                for b in range(NBUF):
                    scat[b].wait()

                    @pl.when(r < NROUND - 1)
                    def _():
                        pltpu.async_copy(
                            x_hbm.at[so.at[b]], rows[b], gsem[b])
                @pl.when(r < NROUND - 2)
                def _():
                    fetch_round(r + 2, sb, db, isem[par])
            return carry

        lax.fori_loop(0, NROUND // 2, body, 0)
        plsc.subcore_barrier()
        pltpu.sync_copy(acc.at[pl.ds(s * RPT, RPT)],
                        out_hbm.at[c].at[pl.ds(s * RPT, RPT)])

    return sc_kernel(x, ei5, zeros_slab)


BN = 2000


def _tc_mlp_body(p_ref, x_ref, w1_ref, b1_ref, w2_ref, b2_ref, e_ref,
                 o_ref):
    scale = 1.0 + e_ref[0, 0]
    y = p_ref[0] + p_ref[1] + scale * x_ref[...]
    h = jnp.maximum(
        jnp.dot(y, w1_ref[...], preferred_element_type=jnp.float32)
        + b1_ref[...], 0.0)
    o_ref[...] = (
        jnp.dot(h, w2_ref[...], preferred_element_type=jnp.float32)
        + b2_ref[...])


def _tc_mlp(partial, x, W1, b1, W2, b2, eps):
    grid = (N // BN,)
    return pl.pallas_call(
        _tc_mlp_body,
        grid=grid,
        in_specs=[
            pl.BlockSpec((NC, BN, D), lambda i: (0, i, 0)),
            pl.BlockSpec((BN, D), lambda i: (i, 0)),
            pl.BlockSpec((D, D), lambda i: (0, 0)),
            pl.BlockSpec((1, D), lambda i: (0, 0)),
            pl.BlockSpec((D, D), lambda i: (0, 0)),
            pl.BlockSpec((1, D), lambda i: (0, 0)),
            pl.BlockSpec((1, 1), lambda i: (0, 0)),
        ],
        out_specs=pl.BlockSpec((BN, D), lambda i: (i, 0)),
        out_shape=jax.ShapeDtypeStruct((N, D), jnp.float32),
    )(partial, x, W1, b1.reshape(1, D), W2, b2.reshape(1, D),
      eps.reshape(1, 1))


def kernel(x, edge_index, W1, b1, W2, b2, eps):
    zeros_slab = jnp.zeros((RPT, D), jnp.float32)
    partial = _sc_partial_segment_sum(x, edge_index.reshape(2 * E), zeros_slab)
    return _tc_mlp(partial, x, W1, b1, W2, b2, eps)


# speedup vs baseline: 1.1017x; 1.0056x over previous
import functools

import jax
import jax.numpy as jnp
from jax import lax
from jax.experimental import pallas as pl
from jax.experimental.pallas import tpu as pltpu
from jax.experimental.pallas import tpu_sc as plsc

N = 10000
E = 320000
D = 128

NC = 2
NS = 16
NW = NC * NS
EPW = E // NW
CH = 40
NCHUNK = EPW // CH
NBUF = 5
NROUND = NCHUNK // NBUF
NP = 10240
RPT = NP // NS


def _sc_partial_segment_sum(x, ei5, zeros_slab):
    mesh = plsc.VectorSubcoreMesh(
        core_axis_name="c", subcore_axis_name="s", num_cores=NC,
        num_subcores=NS)

    @functools.partial(
        pl.kernel,
        out_type=jax.ShapeDtypeStruct((NC, NP, D), jnp.float32),
        mesh=mesh,
        scratch_types=dict(
            sbuf=[pltpu.VMEM((NBUF, CH), jnp.int32) for _ in range(2)],
            dbuf=[pltpu.VMEM((NBUF, CH), jnp.int32) for _ in range(2)],
            rows=[pltpu.VMEM((CH, D), jnp.float32) for _ in range(NBUF)],
            acc=pltpu.VMEM_SHARED((NP, D), jnp.float32),
            gsem=[pltpu.SemaphoreType.DMA for _ in range(NBUF)],
            ssem=[pltpu.SemaphoreType.DMA for _ in range(NBUF)],
            isem=[pltpu.SemaphoreType.DMA for _ in range(2)],
        ),
    )
    def sc_kernel(x_hbm, ei_hbm, zero_hbm, out_hbm,
                  sbuf, dbuf, rows, acc, gsem, ssem, isem):
        c = lax.axis_index("c")
        s = lax.axis_index("s")
        wid = c * NS + s
        ebase = wid * EPW

        def fetch_round(rr, sb, db, sem):
            for b in range(NBUF):
                off = ebase + rr * (NBUF * CH) + b * CH
                pltpu.async_copy(ei_hbm.at[pl.ds(off, CH)],
                                 sb.at[b], sem)
                pltpu.async_copy(ei_hbm.at[pl.ds(E + off, CH)],
                                 db.at[b], sem)

        def drain_round(sb, db, sem):
            for b in range(NBUF):
                pltpu.make_async_copy(ei_hbm.at[pl.ds(ebase, CH)],
                                      sb.at[b], sem).wait()
                pltpu.make_async_copy(ei_hbm.at[pl.ds(ebase, CH)],
                                      db.at[b], sem).wait()

        fetch_round(0, sbuf[0], dbuf[0], isem[0])
        fetch_round(1, sbuf[1], dbuf[1], isem[1])
        drain_round(sbuf[0], dbuf[0], isem[0])
        for b in range(NBUF):
            pltpu.async_copy(x_hbm.at[sbuf[0].at[b]], rows[b], gsem[b])
        pltpu.sync_copy(zero_hbm, acc.at[pl.ds(s * RPT, RPT)])
        plsc.subcore_barrier()

        def body(g, carry):
            for par in range(2):
                r = 2 * g + par
                sb, db = sbuf[par], dbuf[par]
                so, do = sbuf[1 - par], dbuf[1 - par]
                scat = []
                for b in range(NBUF):
                    pltpu.make_async_copy(
                        x_hbm.at[sb.at[b]], rows[b], gsem[b]).wait()
                    scat.append(pltpu.async_copy(
                        rows[b], acc.at[db.at[b]], ssem[b], add=True))
                @pl.when(r < NROUND - 1)
                def _():
                    drain_round(so, do, isem[1 - par])
                for b in range(NBUF):
                    scat[b].wait()

                    @pl.when(r < NROUND - 1)
                    def _():
                        pltpu.async_copy(
                            x_hbm.at[so.at[b]], rows[b], gsem[b])
                @pl.when(r < NROUND - 2)
                def _():
                    fetch_round(r + 2, sb, db, isem[par])
            return carry

        lax.fori_loop(0, NROUND // 2, body, 0)
        plsc.subcore_barrier()
        pltpu.sync_copy(acc.at[pl.ds(s * RPT, RPT)],
                        out_hbm.at[c].at[pl.ds(s * RPT, RPT)])

    return sc_kernel(x, ei5, zeros_slab)


BN = 2000


def _tc_mlp_body(p_ref, x_ref, w1_ref, b1_ref, w2_ref, b2_ref, e_ref,
                 o_ref):
    scale = 1.0 + e_ref[0, 0]
    y = p_ref[0] + p_ref[1] + scale * x_ref[...]
    h = jnp.maximum(
        jnp.dot(y, w1_ref[...], preferred_element_type=jnp.float32)
        + b1_ref[...], 0.0)
    o_ref[...] = (
        jnp.dot(h, w2_ref[...], preferred_element_type=jnp.float32)
        + b2_ref[...])


def _tc_mlp(partial, x, W1, b1, W2, b2, eps):
    grid = (N // BN,)
    return pl.pallas_call(
        _tc_mlp_body,
        grid=grid,
        in_specs=[
            pl.BlockSpec((NC, BN, D), lambda i: (0, i, 0)),
            pl.BlockSpec((BN, D), lambda i: (i, 0)),
            pl.BlockSpec((D, D), lambda i: (0, 0)),
            pl.BlockSpec((1, D), lambda i: (0, 0)),
            pl.BlockSpec((D, D), lambda i: (0, 0)),
            pl.BlockSpec((1, D), lambda i: (0, 0)),
            pl.BlockSpec((1, 1), lambda i: (0, 0)),
        ],
        out_specs=pl.BlockSpec((BN, D), lambda i: (i, 0)),
        out_shape=jax.ShapeDtypeStruct((N, D), jnp.float32),
    )(partial, x, W1, b1.reshape(1, D), W2, b2.reshape(1, D),
      eps.reshape(1, 1))


def kernel(x, edge_index, W1, b1, W2, b2, eps):
    zeros_slab = jnp.zeros((RPT, D), jnp.float32)
    partial = _sc_partial_segment_sum(x, edge_index.reshape(2 * E), zeros_slab)
    return _tc_mlp(partial, x, W1, b1, W2, b2, eps)
